# baseline (speedup 1.0000x reference)
import jax
import jax.numpy as jnp
from jax.experimental import pallas as pl
from jax.experimental.pallas import tpu as pltpu

_IN = 4096
_TB = 256
_KW = 128


def _matvec_relu_kernel(x_ref, w_ref, b_ref, o_ref):
    acc = jnp.zeros((x_ref.shape[0], _KW), jnp.float32)
    for j in range(_IN // _KW):
        acc = acc + x_ref[:, j * _KW:(j + 1) * _KW] * w_ref[:, j * _KW:(j + 1) * _KW]
    s = jnp.sum(acc, axis=-1)
    o_ref[...] = jnp.maximum(s + b_ref[0, 0], 0.0)[None, :]


def kernel(x, weight, bias):
    B = x.shape[0]
    assert x.shape[1] == _IN
    assert B % _TB == 0, "batch must be a multiple of the tile size"
    num_tiles = B // _TB

    bias_smem = jnp.asarray(bias, jnp.float32).reshape(1, 1)

    out = pl.pallas_call(
        _matvec_relu_kernel,
        out_shape=jax.ShapeDtypeStruct((1, B), x.dtype),
        grid=(num_tiles,),
        in_specs=[
            pl.BlockSpec((_TB, _IN), lambda i: (i, 0)),
            pl.BlockSpec((1, _IN), lambda i: (0, 0)),
            pl.BlockSpec(memory_space=pltpu.MemorySpace.SMEM),
        ],
        out_specs=pl.BlockSpec((1, _TB), lambda i: (0, i)),
        compiler_params=pltpu.CompilerParams(
            dimension_semantics=("parallel",),
            vmem_limit_bytes=48 << 20,
        ),
    )(x, weight, bias_smem)

    return out[0].reshape(B, 1)


# speedup vs baseline: 1.1789x; 1.1789x over previous
import jax
import jax.numpy as jnp
from jax.experimental import pallas as pl
from jax.experimental.pallas import tpu as pltpu

_IN = 4096
_TB = 512
_KW = 128


def _matvec_relu_kernel(x_ref, w_ref, b_ref, o_ref):
    tb = x_ref.shape[0]
    half = (_IN // _KW) // 2
    s = None
    for g in (0, 1):
        acc = jnp.zeros((tb, _KW), jnp.float32)
        for j in range(g * half, (g + 1) * half):
            acc = acc + x_ref[:, j * _KW:(j + 1) * _KW] * w_ref[:, j * _KW:(j + 1) * _KW]
        sg = jnp.sum(acc, axis=-1)
        s = sg if s is None else s + sg
    o_ref[...] = jnp.maximum(s + b_ref[0, 0], 0.0).reshape(tb, 1).T


def kernel(x, weight, bias):
    B = x.shape[0]
    assert x.shape[1] == _IN
    assert B % _TB == 0, "batch must be a multiple of the tile size"
    num_tiles = B // _TB

    bias_smem = jnp.asarray(bias, jnp.float32).reshape(1, 1)

    out = pl.pallas_call(
        _matvec_relu_kernel,
        out_shape=jax.ShapeDtypeStruct((1, B), x.dtype),
        grid=(num_tiles,),
        in_specs=[
            pl.BlockSpec((_TB, _IN), lambda i: (i, 0)),
            pl.BlockSpec((1, _IN), lambda i: (0, 0)),
            pl.BlockSpec(memory_space=pltpu.MemorySpace.SMEM),
        ],
        out_specs=pl.BlockSpec((1, _TB), lambda i: (0, i)),
        compiler_params=pltpu.CompilerParams(
            dimension_semantics=("parallel",),
            vmem_limit_bytes=48 << 20,
        ),
    )(x, weight, bias_smem)

    return out[0].reshape(B, 1)
